# baseline (speedup 1.0000x reference)
import functools

import jax
import jax.numpy as jnp
from jax import lax
from jax.experimental import pallas as pl
from jax.experimental.pallas import tpu as pltpu
from jax.experimental.pallas import tpu_sc as plsc

B = 1024
S = 200
D = 64
NC = 2
NS = 16
NW = NC * NS
ROWS = B * S
RPW = ROWS // NW
SPW = B // NW
SR = S // 8
G16 = S // 16


def _sc_embed(ids_flat, tbl3, pos3):
    mesh = plsc.VectorSubcoreMesh(core_axis_name="c", subcore_axis_name="s")

    @functools.partial(
        pl.kernel,
        mesh=mesh,
        out_type=jax.ShapeDtypeStruct((ROWS // 8, 8, D), jnp.float32),
        scratch_types=[
            pltpu.VMEM((RPW,), jnp.int32),
            pltpu.VMEM((SR, 8, D), jnp.float32),
            pltpu.VMEM((SR, 8, D), jnp.float32),
            pltpu.VMEM((SR, 8, D), jnp.float32),
            pltpu.SemaphoreType.DMA,
            pltpu.SemaphoreType.DMA,
            pltpu.SemaphoreType.DMA,
            pltpu.SemaphoreType.DMA,
        ],
    )
    def k(ids_hbm, tok_hbm, pos_hbm, out_hbm, idx_v, pos_v, buf0, buf1,
          g0, g1, o0, o1):
        wid = lax.axis_index("s") * NC + lax.axis_index("c")
        base = wid * RPW
        pltpu.sync_copy(ids_hbm.at[pl.ds(base, RPW)], idx_v)
        pltpu.sync_copy(pos_hbm.at[pl.ds(0, SR)], pos_v)

        def enqueue_rows(s, buf, sem):
            off = s * S

            def fetch16(jt0, vec, n):
                for jj in range(n):
                    i = vec[jj]
                    tid = jax.lax.shift_right_logical(i, 3)
                    srow = jax.lax.bitwise_and(i, 7)
                    pltpu.async_copy(
                        tok_hbm.at[tid, srow],
                        buf.at[jt0 + jj // 8, jj % 8], sem)

            def body(g, carry):
                vec = idx_v[pl.ds(off + g * 16, 16)]
                fetch16(g * 2, vec, 16)
                return carry

            lax.fori_loop(0, G16, body, 0)
            tvec = idx_v[pl.ds(off + G16 * 16, 16)]
            fetch16(G16 * 2, tvec, 8)

        def drain(buf, sem):
            pltpu.make_async_copy(tok_hbm.at[pl.ds(0, SR)], buf, sem).wait()

        def add_pos(buf):
            def body(i, carry):
                for j in range(5):
                    r8 = i * 5 + j
                    for sub in range(8):
                        for kk in range(D // 16):
                            sl = pl.ds(kk * 16, 16)
                            buf[r8, sub, sl] = (
                                buf[r8, sub, sl] + pos_v[r8, sub, sl])
                return carry
            lax.fori_loop(0, SR // 5, body, 0)

        def out_slice(s):
            return out_hbm.at[pl.ds(wid * (SPW * SR) + s * SR, SR)]

        enqueue_rows(0, buf0, g0)
        enqueue_rows(1, buf1, g1)

        def outer(t, carry):
            s0 = t * 2
            drain(buf0, g0)
            add_pos(buf0)
            pltpu.async_copy(buf0, out_slice(s0), o0)
            drain(buf1, g1)
            add_pos(buf1)
            pltpu.async_copy(buf1, out_slice(s0 + 1), o1)
            pltpu.make_async_copy(buf0, out_slice(s0), o0).wait()

            @pl.when(t < (SPW // 2 - 1))
            def _():
                enqueue_rows(s0 + 2, buf0, g0)

            pltpu.make_async_copy(buf1, out_slice(s0 + 1), o1).wait()

            @pl.when(t < (SPW // 2 - 1))
            def _():
                enqueue_rows(s0 + 3, buf1, g1)

            return carry

        lax.fori_loop(0, SPW // 2, outer, 0)

    return k(ids_flat, tbl3, pos3)


def kernel(input_ids, token_table, position_table):
    ids_flat = input_ids.reshape(ROWS)
    tbl3 = token_table.reshape(125000, 8, D)
    pos3 = position_table.reshape(64, 8, D)
    out = _sc_embed(ids_flat, tbl3, pos3)
    return out.reshape(B, S, D)


# speedup vs baseline: 1.1445x; 1.1445x over previous
import functools

import jax
import jax.numpy as jnp
from jax import lax
from jax.experimental import pallas as pl
from jax.experimental.pallas import tpu as pltpu
from jax.experimental.pallas import tpu_sc as plsc

B = 1024
S = 200
D = 64
NC = 2
NS = 16
NW = NC * NS
ROWS = B * S
RPW = ROWS // NW
SPW = B // NW
SR = S // 8
G16 = S // 16


def _sc_embed(ids_flat, tbl3, pos3):
    mesh = plsc.VectorSubcoreMesh(core_axis_name="c", subcore_axis_name="s")

    @functools.partial(
        pl.kernel,
        mesh=mesh,
        out_type=jax.ShapeDtypeStruct((ROWS // 8, 8, D), jnp.float32),
        scratch_types=[
            pltpu.VMEM((RPW,), jnp.int32),
            pltpu.VMEM((SR, 8, D), jnp.float32),
            pltpu.VMEM((SR, 8, D), jnp.float32),
            pltpu.VMEM((SR, 8, D), jnp.float32),
            pltpu.SemaphoreType.DMA,
            pltpu.SemaphoreType.DMA,
            pltpu.SemaphoreType.DMA,
            pltpu.SemaphoreType.DMA,
        ],
    )
    def k(ids_hbm, tok_hbm, pos_hbm, out_hbm, idx_v, pos_v, buf0, buf1,
          g0, g1, o0, o1):
        wid = lax.axis_index("s") * NC + lax.axis_index("c")
        base = wid * RPW
        pltpu.sync_copy(ids_hbm.at[pl.ds(base, RPW)], idx_v)
        pltpu.sync_copy(pos_hbm.at[pl.ds(0, SR)], pos_v)

        def enqueue_rows(s, buf, sem):
            off = s * S

            def fetch16(jt0, vec, n):
                for jj in range(n):
                    i = vec[jj]
                    tid = jax.lax.shift_right_logical(i, 3)
                    srow = jax.lax.bitwise_and(i, 7)
                    pltpu.async_copy(
                        tok_hbm.at[tid, srow],
                        buf.at[jt0 + jj // 8, jj % 8], sem)

            def body(g, carry):
                vec = idx_v[pl.ds(off + g * 16, 16)]
                fetch16(g * 2, vec, 16)
                return carry

            lax.fori_loop(0, G16, body, 0)
            tvec = idx_v[pl.ds(off + G16 * 16, 16)]
            fetch16(G16 * 2, tvec, 8)

        def drain(buf, sem):
            pltpu.make_async_copy(tok_hbm.at[pl.ds(0, SR)], buf, sem).wait()

        def add_pos(buf):
            def body(r8, carry):
                for sub in range(8):
                    for kk in range(D // 16):
                        sl = pl.ds(kk * 16, 16)
                        buf[r8, sub, sl] = buf[r8, sub, sl] + pos_v[r8, sub, sl]
                return carry
            lax.fori_loop(0, SR, body, 0)

        def out_slice(s):
            return out_hbm.at[pl.ds(wid * (SPW * SR) + s * SR, SR)]

        enqueue_rows(0, buf0, g0)
        enqueue_rows(1, buf1, g1)

        def outer(t, carry):
            s0 = t * 2
            drain(buf0, g0)
            add_pos(buf0)
            pltpu.async_copy(buf0, out_slice(s0), o0)
            drain(buf1, g1)
            add_pos(buf1)
            pltpu.async_copy(buf1, out_slice(s0 + 1), o1)
            pltpu.make_async_copy(buf0, out_slice(s0), o0).wait()

            @pl.when(t < (SPW // 2 - 1))
            def _():
                enqueue_rows(s0 + 2, buf0, g0)

            pltpu.make_async_copy(buf1, out_slice(s0 + 1), o1).wait()

            @pl.when(t < (SPW // 2 - 1))
            def _():
                enqueue_rows(s0 + 3, buf1, g1)

            return carry

        lax.fori_loop(0, SPW // 2, outer, 0)

    return k(ids_flat, tbl3, pos3)


def kernel(input_ids, token_table, position_table):
    ids_flat = input_ids.reshape(ROWS)
    tbl3 = token_table.reshape(125000, 8, D)
    pos3 = position_table.reshape(64, 8, D)
    out = _sc_embed(ids_flat, tbl3, pos3)
    return out.reshape(B, S, D)
